# baseline (speedup 1.0000x reference)
import functools

import jax
import jax.numpy as jnp
from jax import lax
from jax.experimental import pallas as pl
from jax.experimental.pallas import tpu as pltpu
from jax.experimental.pallas import tpu_sc as plsc

_R_LO, _R_HI = 1.0, 5.0
_NC = 2
_NS = 16
_NW = _NC * _NS


def _sc_gather(xu, xi, user_table, item_table):
    B = xu.shape[0]
    E = user_table.shape[1]
    b_per_w = B // _NW
    mesh = plsc.VectorSubcoreMesh(core_axis_name="c", subcore_axis_name="s")

    @functools.partial(
        pl.kernel,
        mesh=mesh,
        out_type=[
            jax.ShapeDtypeStruct((B, E), jnp.float32),
            jax.ShapeDtypeStruct((B, E), jnp.float32),
        ],
        scratch_types=[
            pltpu.VMEM((b_per_w,), jnp.int32),
            pltpu.VMEM((b_per_w,), jnp.int32),
            pltpu.VMEM((b_per_w, E), jnp.float32),
            pltpu.VMEM((b_per_w, E), jnp.float32),
            pltpu.SemaphoreType.DMA,
            pltpu.SemaphoreType.DMA,
        ],
        compiler_params=pltpu.CompilerParams(use_tc_tiling_on_sc=False),
    )
    def gather_kernel(xu_hbm, xi_hbm, utab_hbm, itab_hbm, uout_hbm, iout_hbm,
                      uidx_v, iidx_v, urows_v, irows_v, usem, isem):
        wid = lax.axis_index("s") * _NC + lax.axis_index("c")
        base = wid * b_per_w
        pltpu.sync_copy(xu_hbm.at[pl.ds(base, b_per_w)], uidx_v)
        pltpu.sync_copy(xi_hbm.at[pl.ds(base, b_per_w)], iidx_v)
        cu = pltpu.async_copy(utab_hbm.at[uidx_v], urows_v, usem)
        ci = pltpu.async_copy(itab_hbm.at[iidx_v], irows_v, isem)
        cu.wait()
        ci.wait()
        pltpu.sync_copy(urows_v, uout_hbm.at[pl.ds(base, b_per_w)])
        pltpu.sync_copy(irows_v, iout_hbm.at[pl.ds(base, b_per_w)])

    return gather_kernel(xu, xi, user_table, item_table)


def _tc_mlp(X, gu, gi, wg_t, bg2, w1t_u, w1t_i, w1t_g, b12, w2t, b22):
    B = X.shape[0]

    def body(x_ref, gu_ref, gi_ref, wg_ref, bg_ref, w1u_ref, w1i_ref,
             w1g_ref, b1_ref, w2_ref, b2_ref, o_ref):
        g = x_ref[:, 2:].astype(jnp.float32)
        eg = jnp.dot(g, wg_ref[:], preferred_element_type=jnp.float32) + bg_ref[:]
        h = (jnp.dot(gu_ref[:], w1u_ref[:], preferred_element_type=jnp.float32)
             + jnp.dot(gi_ref[:], w1i_ref[:], preferred_element_type=jnp.float32)
             + jnp.dot(eg, w1g_ref[:], preferred_element_type=jnp.float32)
             + b1_ref[:])
        h = jnp.maximum(h, 0.0)
        p = jnp.dot(h, w2_ref[:], preferred_element_type=jnp.float32) + b2_ref[:]
        o_ref[:] = jax.nn.sigmoid(p) * (_R_HI - _R_LO) + _R_LO

    return pl.pallas_call(
        body,
        out_shape=jax.ShapeDtypeStruct((B, 1), jnp.float32),
    )(X, gu, gi, wg_t, bg2, w1t_u, w1t_i, w1t_g, b12, w2t, b22)


def kernel(X, user_table, item_table, Wg, bg, W1, b1, W2, b2):
    xu = X[:, 0]
    xi = X[:, 1]
    gu, gi = _sc_gather(xu, xi, user_table, item_table)

    wg_t = Wg.T
    w1t = W1.T
    w1t_u = w1t[0:16]
    w1t_i = w1t[16:32]
    w1t_g = w1t[32:48]
    w2t = W2.T
    bg2 = bg.reshape(1, -1)
    b12 = b1.reshape(1, -1)
    b22 = b2.reshape(1, -1)
    return _tc_mlp(X, gu, gi, wg_t, bg2, w1t_u, w1t_i, w1t_g, b12, w2t, b22)
